# Initial kernel scaffold; baseline (speedup 1.0000x reference)
import jax
import jax.numpy as jnp
from jax.experimental import pallas as pl


def kernel(x, edge_index, edge_weight, W1, att_src1, att_dst1, b1, W2, att_src2, att_dst2, b2):
    raise NotImplementedError("write your pallas kernel here")


# speedup vs baseline: 1.1014x; 1.1014x over previous
import functools

import jax
import jax.numpy as jnp
from jax.experimental import pallas as pl
from jax.experimental.pallas import tpu as pltpu

N = 10000
E = 320000
NFEAT = 128
NHID = 16
HEADS = 8
NCLASS = 64


def _proj_kernel(x_ref, w_ref, asrc_ref, adst_ref, h_ref, as_ref, ad_ref):
    h = jnp.dot(x_ref[...], w_ref[...], preferred_element_type=jnp.float32)
    h_ref[...] = h
    hh = h.reshape(h.shape[0], -1, asrc_ref.shape[-1])
    as_ref[...] = jnp.sum(hh * asrc_ref[...], axis=-1)
    ad_ref[...] = jnp.sum(hh * adst_ref[...], axis=-1)


def _proj(x, W, att_src, att_dst, heads, out_ch):
    n = x.shape[0]
    blk = 1000
    grid = (n // blk,)
    h, a_s, a_d = pl.pallas_call(
        _proj_kernel,
        grid=grid,
        in_specs=[
            pl.BlockSpec((blk, x.shape[1]), lambda i: (i, 0)),
            pl.BlockSpec((x.shape[1], heads * out_ch), lambda i: (0, 0)),
            pl.BlockSpec((1, heads, out_ch), lambda i: (0, 0, 0)),
            pl.BlockSpec((1, heads, out_ch), lambda i: (0, 0, 0)),
        ],
        out_specs=[
            pl.BlockSpec((blk, heads * out_ch), lambda i: (i, 0)),
            pl.BlockSpec((blk, heads), lambda i: (i, 0)),
            pl.BlockSpec((blk, heads), lambda i: (i, 0)),
        ],
        out_shape=[
            jax.ShapeDtypeStruct((n, heads * out_ch), jnp.float32),
            jax.ShapeDtypeStruct((n, heads), jnp.float32),
            jax.ShapeDtypeStruct((n, heads), jnp.float32),
        ],
    )(x, W, att_src, att_dst)
    return h, a_s, a_d


def _gat_layer(x, src, dst, edge_weight, W, att_src, att_dst, bias, heads, out_ch):
    n = x.shape[0]
    h, a_s, a_d = _proj(x, W, att_src, att_dst, heads, out_ch)
    alpha = a_s[src] + a_d[dst]
    alpha = jax.nn.leaky_relu(alpha, negative_slope=0.2)
    ex = jnp.exp(alpha)
    denom = jax.ops.segment_sum(ex, dst, num_segments=n)
    alpha = ex / (denom[dst] + 1e-16)
    alpha = alpha * edge_weight[:, None]
    hh = h.reshape(n, heads, out_ch)
    msg = hh[src] * alpha[:, :, None]
    out = jax.ops.segment_sum(msg, dst, num_segments=n)
    return out.reshape(n, heads * out_ch) + bias


def kernel(x, edge_index, edge_weight, W1, att_src1, att_dst1, b1, W2, att_src2, att_dst2, b2):
    src = edge_index[0]
    dst = edge_index[1]
    h = _gat_layer(x, src, dst, edge_weight, W1, att_src1, att_dst1, b1, HEADS, NHID)
    h = jax.nn.elu(h)
    h = _gat_layer(h, src, dst, edge_weight, W2, att_src2, att_dst2, b2, 1, NCLASS)
    return jax.nn.log_softmax(h, axis=1)


# speedup vs baseline: 33.7302x; 30.6253x over previous
import functools

import jax
import jax.numpy as jnp
from jax import lax
from jax.experimental import pallas as pl
from jax.experimental.pallas import tpu as pltpu
from jax.experimental.pallas import tpu_sc as plsc

N = 10000
E = 320000
NFEAT = 128
NHID = 16
HEADS = 8
NCLASS = 64

NTILES = 32
EPT = E // NTILES
K1 = 80
NBLK = EPT // K1
NPAD = 10240
RPT = NPAD // 16
W1ROW = 144
A1COLS = 136
W2ROW = 64
A2COLS = 80


def _splat(v, i):
    idx = jnp.full((16, 1), i, jnp.int32)
    return lax.gather(
        v, idx,
        lax.GatherDimensionNumbers(offset_dims=(), collapsed_slice_dims=(0,),
                                   start_index_map=(0,)),
        (1,), mode=lax.GatherScatterMode.PROMISE_IN_BOUNDS)


def _gather16(v, idxvec):
    return lax.gather(
        v, idxvec[:, None],
        lax.GatherDimensionNumbers(offset_dims=(), collapsed_slice_dims=(0,),
                                   start_index_map=(0,)),
        (1,), mode=lax.GatherScatterMode.PROMISE_IN_BOUNDS)


def _proj1_body(x_ref, w_ref, a1s_ref, a1d_ref, hs_ref):
    h = jnp.dot(x_ref[...], w_ref[...], preferred_element_type=jnp.float32)
    a_s = jnp.dot(h, a1s_ref[...], preferred_element_type=jnp.float32)
    a_d = jnp.dot(h, a1d_ref[...], preferred_element_type=jnp.float32)
    hs_ref[...] = jnp.concatenate([h, a_s, a_d], axis=1)


def _proj1(x, W1, A1s, A1d):
    blk = 1000
    return pl.pallas_call(
        _proj1_body,
        grid=(N // blk,),
        in_specs=[
            pl.BlockSpec((blk, NFEAT), lambda i: (i, 0)),
            pl.BlockSpec((NFEAT, 128), lambda i: (0, 0)),
            pl.BlockSpec((128, 8), lambda i: (0, 0)),
            pl.BlockSpec((128, 8), lambda i: (0, 0)),
        ],
        out_specs=pl.BlockSpec((blk, W1ROW), lambda i: (i, 0)),
        out_shape=jax.ShapeDtypeStruct((N, W1ROW), jnp.float32),
    )(x, W1, A1s, A1d)


def _merge1_body(acc_ref, b1_ref, exp8_ref, w2_ref, a2s_ref, a2d_ref,
                 h2_ref, as2_ref, ad2_ref):
    a = acc_ref[0] + acc_ref[1]
    msg = a[:, :128]
    den = a[:, 128:136]
    denx = jnp.dot(den, exp8_ref[...], preferred_element_type=jnp.float32)
    out1 = msg / (denx + 1e-16) + b1_ref[...]
    hmid = jnp.where(out1 > 0, out1, jnp.exp(out1) - 1.0)
    h2 = jnp.dot(hmid, w2_ref[...], preferred_element_type=jnp.float32)
    h2_ref[...] = h2
    as2_ref[...] = jnp.dot(h2, a2s_ref[...], preferred_element_type=jnp.float32)
    ad2_ref[...] = jnp.dot(h2, a2d_ref[...], preferred_element_type=jnp.float32)


def _merge1(accs, b1, EXP8, W2, a2s, a2d):
    blk = 1000
    return pl.pallas_call(
        _merge1_body,
        grid=(N // blk,),
        in_specs=[
            pl.BlockSpec((2, blk, A1COLS + 8), lambda i: (0, i, 0)),
            pl.BlockSpec((1, 128), lambda i: (0, 0)),
            pl.BlockSpec((8, 128), lambda i: (0, 0)),
            pl.BlockSpec((128, NCLASS), lambda i: (0, 0)),
            pl.BlockSpec((NCLASS, 1), lambda i: (0, 0)),
            pl.BlockSpec((NCLASS, 1), lambda i: (0, 0)),
        ],
        out_specs=[
            pl.BlockSpec((blk, NCLASS), lambda i: (i, 0)),
            pl.BlockSpec((blk, 1), lambda i: (i, 0)),
            pl.BlockSpec((blk, 1), lambda i: (i, 0)),
        ],
        out_shape=[
            jax.ShapeDtypeStruct((N, NCLASS), jnp.float32),
            jax.ShapeDtypeStruct((N, 1), jnp.float32),
            jax.ShapeDtypeStruct((N, 1), jnp.float32),
        ],
    )(accs, b1, EXP8, W2, a2s, a2d)


def _final_body(acc_ref, b2_ref, out_ref):
    a = acc_ref[0] + acc_ref[1]
    msg = a[:, :NCLASS]
    den = a[:, NCLASS:NCLASS + 1]
    o = msg / (den + 1e-16) + b2_ref[...]
    m = jnp.max(o, axis=1, keepdims=True)
    s = jnp.log(jnp.sum(jnp.exp(o - m), axis=1, keepdims=True))
    out_ref[...] = o - m - s


def _final(acc2, b2):
    blk = 1000
    return pl.pallas_call(
        _final_body,
        grid=(N // blk,),
        in_specs=[
            pl.BlockSpec((2, blk, A2COLS), lambda i: (0, i, 0)),
            pl.BlockSpec((1, NCLASS), lambda i: (0, 0)),
        ],
        out_specs=pl.BlockSpec((blk, NCLASS), lambda i: (i, 0)),
        out_shape=jax.ShapeDtypeStruct((N, NCLASS), jnp.float32),
    )(acc2, b2)


def _sc1_body(hs_hbm, ad_hbm, src_hbm, dst_hbm, ew_hbm, out_hbm,
              acc, adsh, adb, rowsv, scatv, srcv, dstv, ewv):
    cid = lax.axis_index("c")
    sid = lax.axis_index("s")
    wid = cid * 16 + sid
    lane = lax.iota(jnp.int32, 16)
    sub = lane & 7
    half = lane >> 3
    zero16 = jnp.zeros((16,), jnp.float32)
    cols = [h * 16 + lane for h in range(9)]

    def zrow(r, c):
        rr = jnp.full((16,), r, jnp.int32)
        for j in range(9):
            plsc.store_scatter(scatv, [rr, cols[j]], zero16)
        return c
    lax.fori_loop(0, K1, zrow, 0)
    base = sid * RPT
    for i in range(8):
        pltpu.sync_copy(scatv, acc.at[pl.ds(base + i * 80, 80)])

    @pl.when(sid == 0)
    def _():
        pltpu.sync_copy(ad_hbm, adsh)
    plsc.subcore_barrier()

    def block(b, c):
        off = wid * EPT + b * K1
        pltpu.sync_copy(src_hbm.at[pl.ds(off, K1)], srcv)
        pltpu.sync_copy(dst_hbm.at[pl.ds(off, K1)], dstv)
        pltpu.sync_copy(ew_hbm.at[pl.ds(off, K1)], ewv)
        pltpu.sync_copy(hs_hbm.at[srcv], rowsv)
        pltpu.sync_copy(adsh.at[dstv], adb)

        def pair(p, c2):
            e0 = 2 * p
            epair = e0 + half
            adp = plsc.load_gather(adb, [epair, sub])
            asp = plsc.load_gather(rowsv, [epair, 128 + sub])
            al = asp + adp
            al = jnp.where(al >= 0, al, 0.2 * al)
            exv = jnp.exp(al)
            ewp = plsc.load_gather(ewv, [epair])
            scale = exv * ewp
            for t in range(2):
                e = jnp.full((16,), e0 + t, jnp.int32)
                for h in range(8):
                    s = _splat(scale, t * 8 + h)
                    hv = plsc.load_gather(rowsv, [e, cols[h]])
                    plsc.store_scatter(scatv, [e, cols[h]], hv * s)
                ext = exv if t == 0 else _gather16(exv, sub + 8)
                plsc.store_scatter(scatv, [e, cols[8]],
                                   jnp.where(lane < 8, ext, 0.0))
            return c2
        lax.fori_loop(0, K1 // 2, pair, 0)
        pltpu.sync_copy(scatv, acc.at[dstv], add=True)
        return c
    lax.fori_loop(0, NBLK, block, 0)
    plsc.subcore_barrier()
    pltpu.sync_copy(acc.at[pl.ds(base, RPT)], out_hbm.at[cid, pl.ds(base, RPT)])


def _sc_layer1(hs, ad, src, dst, ew):
    mesh = plsc.VectorSubcoreMesh(core_axis_name="c", subcore_axis_name="s")
    f = pl.kernel(
        _sc1_body,
        out_type=jax.ShapeDtypeStruct((2, NPAD, W1ROW), jnp.float32),
        mesh=mesh,
        compiler_params=pltpu.CompilerParams(use_tc_tiling_on_sc=False, needs_layout_passes=False),
        scratch_types=[
            pltpu.VMEM_SHARED((NPAD, W1ROW), jnp.float32),
            pltpu.VMEM_SHARED((N, 8), jnp.float32),
            pltpu.VMEM((K1, 8), jnp.float32),
            pltpu.VMEM((K1, W1ROW), jnp.float32),
            pltpu.VMEM((K1, W1ROW), jnp.float32),
            pltpu.VMEM((K1,), jnp.int32),
            pltpu.VMEM((K1,), jnp.int32),
            pltpu.VMEM((K1,), jnp.float32),
        ],
    )
    return f(hs, ad, src, dst, ew)


def _sc2_body(h2_hbm, as2_hbm, ad2_hbm, src_hbm, dst_hbm, ew_hbm, out_hbm,
              acc, as2v, ad2v, rowsv, scatv, srcv, dstv, ewv):
    cid = lax.axis_index("c")
    sid = lax.axis_index("s")
    wid = cid * 16 + sid
    lane = lax.iota(jnp.int32, 16)
    zero16 = jnp.zeros((16,), jnp.float32)
    cols = [c * 16 + lane for c in range(5)]

    def zrow(r, c):
        rr = jnp.full((16,), r, jnp.int32)
        for j in range(5):
            plsc.store_scatter(scatv, [rr, cols[j]], zero16)
        return c
    lax.fori_loop(0, K1, zrow, 0)
    base = sid * RPT
    for i in range(8):
        pltpu.sync_copy(scatv, acc.at[pl.ds(base + i * 80, 80)])

    pltpu.sync_copy(as2_hbm, as2v)
    pltpu.sync_copy(ad2_hbm, ad2v)
    plsc.subcore_barrier()

    def block(b, c):
        off = wid * EPT + b * K1
        pltpu.sync_copy(src_hbm.at[pl.ds(off, K1)], srcv)
        pltpu.sync_copy(dst_hbm.at[pl.ds(off, K1)], dstv)
        pltpu.sync_copy(ew_hbm.at[pl.ds(off, K1)], ewv)
        pltpu.sync_copy(h2_hbm.at[srcv], rowsv)

        def group(g, c2):
            idx16 = g * 16 + lane
            s16 = plsc.load_gather(srcv, [idx16])
            d16 = plsc.load_gather(dstv, [idx16])
            al = plsc.load_gather(as2v, [s16]) + plsc.load_gather(ad2v, [d16])
            al = jnp.where(al >= 0, al, 0.2 * al)
            exv = jnp.exp(al)
            scale = exv * plsc.load_gather(ewv, [idx16])
            for j in range(16):
                e = jnp.full((16,), g * 16 + j, jnp.int32)
                sj = _splat(scale, j)
                for c4 in range(4):
                    hv = plsc.load_gather(rowsv, [e, cols[c4]])
                    plsc.store_scatter(scatv, [e, cols[c4]], hv * sj)
                exj = _splat(exv, j)
                plsc.store_scatter(scatv, [e, cols[4]],
                                   jnp.where(lane < 1, exj, 0.0))
            return c2
        lax.fori_loop(0, K1 // 16, group, 0)
        pltpu.sync_copy(scatv, acc.at[dstv], add=True)
        return c
    lax.fori_loop(0, NBLK, block, 0)
    plsc.subcore_barrier()
    pltpu.sync_copy(acc.at[pl.ds(base, RPT)], out_hbm.at[cid, pl.ds(base, RPT)])


def _sc_layer2(h2, as2, ad2, src, dst, ew):
    mesh = plsc.VectorSubcoreMesh(core_axis_name="c", subcore_axis_name="s")
    f = pl.kernel(
        _sc2_body,
        out_type=jax.ShapeDtypeStruct((2, NPAD, A2COLS), jnp.float32),
        mesh=mesh,
        compiler_params=pltpu.CompilerParams(use_tc_tiling_on_sc=False, needs_layout_passes=False),
        scratch_types=[
            pltpu.VMEM_SHARED((NPAD, A2COLS), jnp.float32),
            pltpu.VMEM((N,), jnp.float32),
            pltpu.VMEM((N,), jnp.float32),
            pltpu.VMEM((K1, W2ROW), jnp.float32),
            pltpu.VMEM((K1, A2COLS), jnp.float32),
            pltpu.VMEM((K1,), jnp.int32),
            pltpu.VMEM((K1,), jnp.int32),
            pltpu.VMEM((K1,), jnp.float32),
        ],
    )
    return f(h2, as2, ad2, src, dst, ew)


def kernel(x, edge_index, edge_weight, W1, att_src1, att_dst1, b1,
           W2, att_src2, att_dst2, b2):
    src = edge_index[0]
    dst = edge_index[1]
    eye8 = jnp.eye(8, dtype=jnp.float32)
    A1s = (eye8[:, None, :] * att_src1.reshape(8, 16)[:, :, None]).reshape(128, 8)
    A1d = (eye8[:, None, :] * att_dst1.reshape(8, 16)[:, :, None]).reshape(128, 8)
    EXP8 = jnp.repeat(eye8, 16, axis=1)

    hs = _proj1(x, W1, A1s, A1d)
    accs = _sc_layer1(hs, hs[:, 136:144], src, dst, edge_weight)
    h2, as2, ad2 = _merge1(accs, b1.reshape(1, 128), EXP8, W2,
                           att_src2.reshape(NCLASS, 1), att_dst2.reshape(NCLASS, 1))
    acc2 = _sc_layer2(h2, as2.reshape(N), ad2.reshape(N), src, dst, edge_weight)
    return _final(acc2, b2.reshape(1, NCLASS))


# speedup vs baseline: 107.4573x; 3.1858x over previous
import jax
import jax.numpy as jnp
from jax import lax
from jax.experimental import pallas as pl
from jax.experimental.pallas import tpu as pltpu
from jax.experimental.pallas import tpu_sc as plsc

N = 10000
E = 320000
NFEAT = 128
NHID = 16
HEADS = 8
NCLASS = 64

NTILES = 32
EPT = E // NTILES
K1 = 80
NBLK = EPT // K1
NPAD = 10240
RPT = NPAD // 16
W1ROW = 144
W2ROW = 80


def _splat(v, i):
    idx = jnp.full((16, 1), i, jnp.int32)
    return lax.gather(
        v, idx,
        lax.GatherDimensionNumbers(offset_dims=(), collapsed_slice_dims=(0,),
                                   start_index_map=(0,)),
        (1,), mode=lax.GatherScatterMode.PROMISE_IN_BOUNDS)


def _gather16(v, idxvec):
    return lax.gather(
        v, idxvec[:, None],
        lax.GatherDimensionNumbers(offset_dims=(), collapsed_slice_dims=(0,),
                                   start_index_map=(0,)),
        (1,), mode=lax.GatherScatterMode.PROMISE_IN_BOUNDS)


def _proj1_body(x_ref, w_ref, a1s_ref, a1d_ref, hs_ref):
    h = jnp.dot(x_ref[...], w_ref[...], preferred_element_type=jnp.float32)
    a_s = jnp.dot(h, a1s_ref[...], preferred_element_type=jnp.float32)
    a_d = jnp.dot(h, a1d_ref[...], preferred_element_type=jnp.float32)
    hs_ref[...] = jnp.concatenate([h, a_s, a_d], axis=1)


def _proj1(x, W1, A1s, A1d):
    blk = 1000
    return pl.pallas_call(
        _proj1_body,
        grid=(N // blk,),
        in_specs=[
            pl.BlockSpec((blk, NFEAT), lambda i: (i, 0)),
            pl.BlockSpec((NFEAT, 128), lambda i: (0, 0)),
            pl.BlockSpec((128, 8), lambda i: (0, 0)),
            pl.BlockSpec((128, 8), lambda i: (0, 0)),
        ],
        out_specs=pl.BlockSpec((blk, W1ROW), lambda i: (i, 0)),
        out_shape=jax.ShapeDtypeStruct((N, W1ROW), jnp.float32),
    )(x, W1, A1s, A1d)


def _merge1_body(acc_ref, b1_ref, exp8_ref, w2_ref, a2s_ref, a2d_ref, zpad_ref,
                 h2_ref, ad2_ref):
    a = acc_ref[0] + acc_ref[1]
    msg = a[:, :128]
    den = a[:, 128:136]
    denx = jnp.dot(den, exp8_ref[...], preferred_element_type=jnp.float32)
    out1 = msg / (denx + 1e-16) + b1_ref[...]
    hmid = jnp.where(out1 > 0, out1, jnp.exp(out1) - 1.0)
    h2 = jnp.dot(hmid, w2_ref[...], preferred_element_type=jnp.float32)
    as2 = jnp.dot(h2, a2s_ref[...], preferred_element_type=jnp.float32)
    ad2 = jnp.dot(h2, a2d_ref[...], preferred_element_type=jnp.float32)
    h2_ref[...] = jnp.concatenate([h2, as2, ad2, zpad_ref[...]], axis=1)
    ad2_ref[...] = ad2


def _merge1(accs, b1, EXP8, W2, a2s, a2d):
    blk = 1000
    zpad = jnp.zeros((blk, 14), jnp.float32)
    return pl.pallas_call(
        _merge1_body,
        grid=(N // blk,),
        in_specs=[
            pl.BlockSpec((2, blk, W1ROW), lambda i: (0, i, 0)),
            pl.BlockSpec((1, 128), lambda i: (0, 0)),
            pl.BlockSpec((8, 128), lambda i: (0, 0)),
            pl.BlockSpec((128, NCLASS), lambda i: (0, 0)),
            pl.BlockSpec((NCLASS, 1), lambda i: (0, 0)),
            pl.BlockSpec((NCLASS, 1), lambda i: (0, 0)),
            pl.BlockSpec((blk, 14), lambda i: (0, 0)),
        ],
        out_specs=[
            pl.BlockSpec((blk, W2ROW), lambda i: (i, 0)),
            pl.BlockSpec((blk, 1), lambda i: (i, 0)),
        ],
        out_shape=[
            jax.ShapeDtypeStruct((N, W2ROW), jnp.float32),
            jax.ShapeDtypeStruct((N, 1), jnp.float32),
        ],
    )(accs, b1, EXP8, W2, a2s, a2d, zpad)


def _final_body(acc_ref, b2_ref, out_ref):
    a = acc_ref[0] + acc_ref[1]
    msg = a[:, :NCLASS]
    den = a[:, NCLASS:NCLASS + 1]
    o = msg / (den + 1e-16) + b2_ref[...]
    m = jnp.max(o, axis=1, keepdims=True)
    s = jnp.log(jnp.sum(jnp.exp(o - m), axis=1, keepdims=True))
    out_ref[...] = o - m - s


def _final(acc2, b2):
    blk = 1000
    return pl.pallas_call(
        _final_body,
        grid=(N // blk,),
        in_specs=[
            pl.BlockSpec((2, blk, W2ROW), lambda i: (0, i, 0)),
            pl.BlockSpec((1, NCLASS), lambda i: (0, 0)),
        ],
        out_specs=pl.BlockSpec((blk, NCLASS), lambda i: (i, 0)),
        out_shape=jax.ShapeDtypeStruct((N, NCLASS), jnp.float32),
    )(acc2, b2)


def _zero_rows(rows, ncol16):
    lane = lax.iota(jnp.int32, 16)
    zero16 = jnp.zeros((16,), jnp.float32)
    cols = [c * 16 + lane for c in range(ncol16)]

    def zrow(r, c):
        rr = jnp.full((16,), r, jnp.int32)
        for j in range(ncol16):
            plsc.store_scatter(rows, [rr, cols[j]], zero16)
        return c
    lax.fori_loop(0, K1, zrow, 0)


def _init_acc(rows0, acc, sid):
    base = sid * RPT
    for i in range(8):
        pltpu.sync_copy(rows0, acc.at[pl.ds(base + i * K1, K1)])


def _copy_dst(idxr, dstr):
    for q in range(K1 // 16):
        dstr[pl.ds(q * 16, 16)] = idxr[1, pl.ds(q * 16, 16)]


def _sc1_body(hs_hbm, ad_hbm, eidx_hbm, out_hbm,
              acc, adsh, rows0, rows1, adb0, adb1, idx0, idx1, dst0, dst1,
              gs0, gs1, as0, as1, is0, is1, ss0, ss1):
    cid = lax.axis_index("c")
    sid = lax.axis_index("s")
    wid = cid * 16 + sid
    ebase = wid * EPT
    lane = lax.iota(jnp.int32, 16)
    sub = lane & 7
    half = lane >> 3
    f2 = jnp.full((16,), 2, jnp.int32)
    rows = (rows0, rows1)
    adb = (adb0, adb1)
    idxb = (idx0, idx1)
    dstb = (dst0, dst1)
    gs = (gs0, gs1)
    asem = (as0, as1)
    isem = (is0, is1)
    ssem = (ss0, ss1)

    _zero_rows(rows0, 9)
    _init_acc(rows0, acc, sid)

    @pl.when(sid == 0)
    def _():
        pltpu.sync_copy(ad_hbm, adsh)
    plsc.subcore_barrier()

    def start_idx(b, par):
        pltpu.async_copy(eidx_hbm.at[:, pl.ds(ebase + b * K1, K1)],
                         idxb[par], isem[par])

    def wait_idx(par):
        pltpu.make_async_copy(eidx_hbm.at[:, pl.ds(0, K1)],
                              idxb[par], isem[par]).wait()

    def start_gathers(par):
        _copy_dst(idxb[par], dstb[par])
        pltpu.async_copy(hs_hbm.at[idxb[par].at[0]], rows[par], gs[par])
        pltpu.async_copy(adsh.at[dstb[par]], adb[par], asem[par])

    def wait_gathers(par):
        pltpu.make_async_copy(hs_hbm.at[idxb[par].at[0]], rows[par],
                              gs[par]).wait()
        pltpu.make_async_copy(adsh.at[dstb[par]], adb[par], asem[par]).wait()

    def wait_scatter(par):
        pltpu.make_async_copy(rows[par], acc.at[dstb[par]], ssem[par]).wait()

    def compute(par):
        rowsr = rows[par]
        adbr = adb[par]
        idxr = idxb[par]

        def pair(p, c2):
            e0 = 2 * p
            epair = e0 + half
            adp = plsc.load_gather(adbr, [epair, sub])
            asp = plsc.load_gather(rowsr, [epair, 128 + sub])
            al = asp + adp
            al = jnp.maximum(al, 0.2 * al)
            exv = jnp.exp(al)
            ewp = plsc.bitcast(plsc.load_gather(idxr, [f2, epair]), jnp.float32)
            scale = exv * ewp
            for t in range(2):
                e = e0 + t
                for h in range(8):
                    s = _splat(scale, t * 8 + h)
                    rowsr[e, pl.ds(h * 16, 16)] = rowsr[e, pl.ds(h * 16, 16)] * s
                ext = exv if t == 0 else _gather16(exv, sub + 8)
                rowsr[e, pl.ds(128, 16)] = jnp.where(lane < 8, ext, 0.0)
            return c2
        lax.fori_loop(0, K1 // 2, pair, 0)

    start_idx(0, 0)
    wait_idx(0)
    start_gathers(0)
    start_idx(1, 1)

    def iteration(i, c):
        for par in (0, 1):
            nxt = 1 - par
            b = 2 * i + par
            wait_gathers(par)
            wait_idx(nxt)
            if par == 0:
                @pl.when(i > 0)
                def _():
                    wait_scatter(nxt)
            else:
                wait_scatter(nxt)
            start_gathers(nxt)
            if par == 0:
                start_idx(b + 2, par)
            else:
                @pl.when(i < (NBLK - 1) // 2 - 1)
                def _():
                    start_idx(b + 2, par)
            compute(par)
            pltpu.async_copy(rows[par], acc.at[dstb[par]], ssem[par], add=True)
        return c
    lax.fori_loop(0, (NBLK - 1) // 2, iteration, 0)

    wait_gathers(0)
    compute(0)
    pltpu.sync_copy(rows[0], acc.at[dstb[0]], add=True)
    wait_scatter(1)
    plsc.subcore_barrier()
    base = sid * RPT
    pltpu.sync_copy(acc.at[pl.ds(base, RPT)], out_hbm.at[cid, pl.ds(base, RPT)])


def _sc_layer1(hs, ad, eidx):
    mesh = plsc.VectorSubcoreMesh(core_axis_name="c", subcore_axis_name="s")
    f = pl.kernel(
        _sc1_body,
        out_type=jax.ShapeDtypeStruct((2, NPAD, W1ROW), jnp.float32),
        mesh=mesh,
        compiler_params=pltpu.CompilerParams(use_tc_tiling_on_sc=False,
                                             needs_layout_passes=False),
        scratch_types=[
            pltpu.VMEM_SHARED((NPAD, W1ROW), jnp.float32),
            pltpu.VMEM_SHARED((N, 8), jnp.float32),
            pltpu.VMEM((K1, W1ROW), jnp.float32),
            pltpu.VMEM((K1, W1ROW), jnp.float32),
            pltpu.VMEM((K1, 8), jnp.float32),
            pltpu.VMEM((K1, 8), jnp.float32),
            pltpu.VMEM((3, K1), jnp.int32),
            pltpu.VMEM((3, K1), jnp.int32),
            pltpu.VMEM((K1,), jnp.int32),
            pltpu.VMEM((K1,), jnp.int32),
        ] + [pltpu.SemaphoreType.DMA] * 8,
    )
    return f(hs, ad, eidx)


def _sc2_body(h2_hbm, ad2_hbm, eidx_hbm, out_hbm,
              acc, ad2v, rows0, rows1, idx0, idx1, dst0, dst1,
              gs0, gs1, is0, is1, ss0, ss1):
    cid = lax.axis_index("c")
    sid = lax.axis_index("s")
    wid = cid * 16 + sid
    ebase = wid * EPT
    lane = lax.iota(jnp.int32, 16)
    f1 = jnp.full((16,), 1, jnp.int32)
    f2 = jnp.full((16,), 2, jnp.int32)
    f64 = jnp.full((16,), 64, jnp.int32)
    rows = (rows0, rows1)
    idxb = (idx0, idx1)
    dstb = (dst0, dst1)
    gs = (gs0, gs1)
    isem = (is0, is1)
    ssem = (ss0, ss1)

    _zero_rows(rows0, 5)
    _init_acc(rows0, acc, sid)
    pltpu.sync_copy(ad2_hbm, ad2v)
    plsc.subcore_barrier()

    def start_idx(b, par):
        pltpu.async_copy(eidx_hbm.at[:, pl.ds(ebase + b * K1, K1)],
                         idxb[par], isem[par])

    def wait_idx(par):
        pltpu.make_async_copy(eidx_hbm.at[:, pl.ds(0, K1)],
                              idxb[par], isem[par]).wait()

    def start_gathers(par):
        _copy_dst(idxb[par], dstb[par])
        pltpu.async_copy(h2_hbm.at[idxb[par].at[0]], rows[par], gs[par])

    def wait_gathers(par):
        pltpu.make_async_copy(h2_hbm.at[idxb[par].at[0]], rows[par],
                              gs[par]).wait()

    def wait_scatter(par):
        pltpu.make_async_copy(rows[par], acc.at[dstb[par]], ssem[par]).wait()

    def compute(par):
        rowsr = rows[par]
        idxr = idxb[par]
        for g in range(K1 // 16):
            idx16 = g * 16 + lane
            as16 = plsc.load_gather(rowsr, [idx16, f64])
            d16 = plsc.load_gather(idxr, [f1, idx16])
            ad16 = plsc.load_gather(ad2v, [d16])
            al = as16 + ad16
            al = jnp.maximum(al, 0.2 * al)
            exv = jnp.exp(al)
            ew16 = plsc.bitcast(plsc.load_gather(idxr, [f2, idx16]), jnp.float32)
            scale = exv * ew16
            for j in range(16):
                e = g * 16 + j
                sj = _splat(scale, j)
                for c in range(4):
                    rowsr[e, pl.ds(c * 16, 16)] = rowsr[e, pl.ds(c * 16, 16)] * sj
                exj = _splat(exv, j)
                rowsr[e, pl.ds(64, 16)] = jnp.where(lane < 1, exj, 0.0)

    start_idx(0, 0)
    wait_idx(0)
    start_gathers(0)
    start_idx(1, 1)

    def iteration(i, c):
        for par in (0, 1):
            nxt = 1 - par
            b = 2 * i + par
            wait_gathers(par)
            wait_idx(nxt)
            if par == 0:
                @pl.when(i > 0)
                def _():
                    wait_scatter(nxt)
            else:
                wait_scatter(nxt)
            start_gathers(nxt)
            if par == 0:
                start_idx(b + 2, par)
            else:
                @pl.when(i < (NBLK - 1) // 2 - 1)
                def _():
                    start_idx(b + 2, par)
            compute(par)
            pltpu.async_copy(rows[par], acc.at[dstb[par]], ssem[par], add=True)
        return c
    lax.fori_loop(0, (NBLK - 1) // 2, iteration, 0)

    wait_gathers(0)
    compute(0)
    pltpu.sync_copy(rows[0], acc.at[dstb[0]], add=True)
    wait_scatter(1)
    plsc.subcore_barrier()
    base = sid * RPT
    pltpu.sync_copy(acc.at[pl.ds(base, RPT)], out_hbm.at[cid, pl.ds(base, RPT)])


def _sc_layer2(h2, ad2, eidx):
    mesh = plsc.VectorSubcoreMesh(core_axis_name="c", subcore_axis_name="s")
    f = pl.kernel(
        _sc2_body,
        out_type=jax.ShapeDtypeStruct((2, NPAD, W2ROW), jnp.float32),
        mesh=mesh,
        compiler_params=pltpu.CompilerParams(use_tc_tiling_on_sc=False,
                                             needs_layout_passes=False),
        scratch_types=[
            pltpu.VMEM_SHARED((NPAD, W2ROW), jnp.float32),
            pltpu.VMEM((N,), jnp.float32),
            pltpu.VMEM((K1, W2ROW), jnp.float32),
            pltpu.VMEM((K1, W2ROW), jnp.float32),
            pltpu.VMEM((3, K1), jnp.int32),
            pltpu.VMEM((3, K1), jnp.int32),
            pltpu.VMEM((K1,), jnp.int32),
            pltpu.VMEM((K1,), jnp.int32),
        ] + [pltpu.SemaphoreType.DMA] * 6,
    )
    return f(h2, ad2, eidx)


def kernel(x, edge_index, edge_weight, W1, att_src1, att_dst1, b1,
           W2, att_src2, att_dst2, b2):
    src = edge_index[0]
    dst = edge_index[1]
    eidx = jnp.stack([src, dst, lax.bitcast_convert_type(edge_weight, jnp.int32)])
    eye8 = jnp.eye(8, dtype=jnp.float32)
    A1s = (eye8[:, None, :] * att_src1.reshape(8, 16)[:, :, None]).reshape(128, 8)
    A1d = (eye8[:, None, :] * att_dst1.reshape(8, 16)[:, :, None]).reshape(128, 8)
    EXP8 = jnp.repeat(eye8, 16, axis=1)

    hs = _proj1(x, W1, A1s, A1d)
    accs = _sc_layer1(hs, hs[:, 136:144], eidx)
    h2aug, ad2 = _merge1(accs, b1.reshape(1, 128), EXP8, W2,
                         att_src2.reshape(NCLASS, 1), att_dst2.reshape(NCLASS, 1))
    acc2 = _sc_layer2(h2aug, ad2.reshape(N), eidx)
    return _final(acc2, b2.reshape(1, NCLASS))
